# Initial kernel scaffold; baseline (speedup 1.0000x reference)
import jax
import jax.numpy as jnp
from jax.experimental import pallas as pl


def kernel(x, edge_index, W1, b1, W2, b2, Wh, bh):
    raise NotImplementedError("write your pallas kernel here")


# speedup vs baseline: 10.3314x; 10.3314x over previous
import functools

import jax
import jax.numpy as jnp
from jax import lax
from jax.experimental import pallas as pl
from jax.experimental.pallas import tpu as pltpu
from jax.experimental.pallas import tpu_sc as plsc

N = 10000
NPAD = 10240
E = 320000
D = 128
NC, NS = 2, 16
TILES = NC * NS
BLK = 128
BPT = 79
EPAD = TILES * BPT * BLK
ROWS_PER_TILE = NPAD // NS


def _sc_mesh():
    return plsc.VectorSubcoreMesh(core_axis_name="c", subcore_axis_name="s")


def _deg_kernel_body(dst_hbm, out_hbm, hist_v, idx_v):
    c = lax.axis_index("c")
    s = lax.axis_index("s")
    wid = c * NS + s

    @pl.loop(0, NPAD // 16)
    def _(i):
        hist_v[pl.ds(i * 16, 16)] = jnp.zeros((16,), jnp.float32)

    base0 = wid * BPT * BLK

    @pl.loop(0, BPT)
    def _(b):
        pltpu.sync_copy(dst_hbm.at[pl.ds(base0 + b * BLK, BLK)], idx_v)
        for j in range(BLK // 16):
            idx = idx_v[pl.ds(j * 16, 16)]
            cnt, last = plsc.scan_count(idx)
            plsc.addupdate_scatter(hist_v, [idx], cnt.astype(jnp.float32),
                                   mask=last)

    pltpu.sync_copy(hist_v, out_hbm.at[pl.ds(wid * NPAD, NPAD)])


@jax.jit
def _deg(dst_pad):
    fn = pl.kernel(
        _deg_kernel_body,
        out_type=jax.ShapeDtypeStruct((TILES * NPAD,), jnp.float32),
        mesh=_sc_mesh(),
        compiler_params=pltpu.CompilerParams(needs_layout_passes=False),
        scratch_types=[
            pltpu.VMEM((NPAD,), jnp.float32),
            pltpu.VMEM((BLK,), jnp.int32),
        ],
    )
    return fn(dst_pad)


def _agg_kernel_body(g_hbm, src_hbm, dst_hbm, zeros_hbm, out_hbm,
                     acc, sidx_v, didx_v, rows_v, sem):
    c = lax.axis_index("c")
    s = lax.axis_index("s")
    wid = c * NS + s
    row0 = s * ROWS_PER_TILE

    @pl.when(c == 0)
    def _():
        @pl.loop(0, ROWS_PER_TILE // BLK)
        def _(k):
            pltpu.sync_copy(g_hbm.at[pl.ds(row0 + k * BLK, BLK)], rows_v)
            pltpu.sync_copy(rows_v, acc.at[pl.ds(row0 + k * BLK, BLK)])

    @pl.when(c == 1)
    def _():
        @pl.loop(0, ROWS_PER_TILE // BLK)
        def _(k):
            pltpu.sync_copy(zeros_hbm, rows_v)
            pltpu.sync_copy(rows_v, acc.at[pl.ds(row0 + k * BLK, BLK)])

    plsc.subcore_barrier()

    base0 = wid * BPT * BLK

    @pl.loop(0, BPT)
    def _(b):
        pltpu.sync_copy(src_hbm.at[pl.ds(base0 + b * BLK, BLK)], sidx_v)
        pltpu.async_copy(g_hbm.at[sidx_v], rows_v, sem).wait()
        pltpu.sync_copy(dst_hbm.at[pl.ds(base0 + b * BLK, BLK)], didx_v)
        pltpu.sync_copy(rows_v, acc.at[didx_v], add=True)

    plsc.subcore_barrier()

    @pl.loop(0, ROWS_PER_TILE // BLK)
    def _(k):
        pltpu.sync_copy(acc.at[pl.ds(row0 + k * BLK, BLK)], rows_v)
        pltpu.sync_copy(rows_v, out_hbm.at[pl.ds(c * NPAD + row0 + k * BLK, BLK)])


@jax.jit
def _agg(g, src_pad, dst_pad, zeros_row):
    fn = pl.kernel(
        _agg_kernel_body,
        out_type=jax.ShapeDtypeStruct((NC * NPAD, D), jnp.float32),
        mesh=_sc_mesh(),
        scratch_types=[
            pltpu.VMEM_SHARED((NPAD, D), jnp.float32),
            pltpu.VMEM((BLK,), jnp.int32),
            pltpu.VMEM((BLK,), jnp.int32),
            pltpu.VMEM((BLK, D), jnp.float32),
            pltpu.SemaphoreType.DMA,
        ],
    )
    return fn(g, src_pad, dst_pad, zeros_row)


def _dinv_from(degp_ref):
    deg = jnp.sum(degp_ref[...], axis=0) + 1.0
    dinv = lax.rsqrt(deg)
    rows = lax.iota(jnp.int32, NPAD)
    dinv = jnp.where(rows < N, dinv, 0.0)
    return dinv.reshape(NPAD, 1)


def _tc1_body(x_ref, w_ref, degp_ref, o_ref):
    dinv = _dinv_from(degp_ref)
    h = jnp.dot(x_ref[...], w_ref[...], preferred_element_type=jnp.float32)
    o_ref[...] = h * dinv


def _tc2_body(p_ref, degp_ref, b_ref, w_ref, o_ref):
    dinv = _dinv_from(degp_ref)
    agg = p_ref[0:NPAD, :] + p_ref[NPAD:2 * NPAD, :]
    a = agg * dinv + b_ref[...]
    r = jnp.maximum(a, 0.0)
    h = jnp.dot(r, w_ref[...], preferred_element_type=jnp.float32)
    o_ref[...] = h * dinv


def _tc3_body(p_ref, degp_ref, b_ref, wh_ref, bh_ref, o_ref):
    dinv = _dinv_from(degp_ref)
    agg = p_ref[0:NPAD, :] + p_ref[NPAD:2 * NPAD, :]
    a = agg * dinv + b_ref[...]
    r = jnp.maximum(a, 0.0)
    out = jnp.dot(r, wh_ref[...], preferred_element_type=jnp.float32) + bh_ref[...]
    o_ref[...] = out[0:N, :]


def _tc_call(body, out_shape, *args):
    return pl.pallas_call(
        body,
        out_shape=out_shape,
    )(*args)


def kernel(x, edge_index, W1, b1, W2, b2, Wh, bh):
    src = edge_index[0].astype(jnp.int32)
    dst = edge_index[1].astype(jnp.int32)
    padlen = EPAD - E
    padidx = jnp.full((padlen,), N, dtype=jnp.int32)
    src_pad = jnp.concatenate([src, padidx])
    dst_pad = jnp.concatenate([dst, padidx])

    x_ext = jnp.concatenate([x, jnp.zeros((NPAD - N, D), jnp.float32)], axis=0)
    zeros_row = jnp.zeros((BLK, D), jnp.float32)

    degp = _deg(dst_pad).reshape(TILES, NPAD)

    g1 = _tc_call(_tc1_body, jax.ShapeDtypeStruct((NPAD, D), jnp.float32),
                  x_ext, W1, degp)
    p1 = _agg(g1, src_pad, dst_pad, zeros_row)
    g2 = _tc_call(_tc2_body, jax.ShapeDtypeStruct((NPAD, D), jnp.float32),
                  p1, degp, b1.reshape(1, D), W2)
    p2 = _agg(g2, src_pad, dst_pad, zeros_row)
    out = _tc_call(_tc3_body, jax.ShapeDtypeStruct((N, bh.shape[0]), jnp.float32),
                   p2, degp, b2.reshape(1, D), Wh, bh.reshape(1, -1))
    return out
